# baseline (speedup 1.0000x reference)
import functools

import jax
import jax.numpy as jnp
from jax import lax
from jax.experimental import pallas as pl
from jax.experimental.pallas import tpu as pltpu
from jax.experimental.pallas import tpu_sc as plsc

N_NODES = 10000
D_IN = 128
D_HID = 300
D_OUT = 2

NPAD = 10240
NC = 2
NS = 16
NW = NC * NS
CHUNK = 128
EC = 79
TOT_E = NW * EC * CHUNK
SB = NPAD // NS

D_HID_PAD = 384
D_OUT_PAD = 128

_mesh = plsc.VectorSubcoreMesh(core_axis_name="c", subcore_axis_name="s")


@functools.partial(
    pl.kernel,
    out_type=jax.ShapeDtypeStruct((NC, NPAD, 128), jnp.float32),
    mesh=_mesh,
    scratch_types=[
        pltpu.VMEM((EC, CHUNK), jnp.int32),
        pltpu.VMEM((CHUNK, 128), jnp.float32),
        pltpu.VMEM_SHARED((NPAD, 128), jnp.float32),
    ],
)
def _deg_kernel(dst_hbm, ones_hbm, zeros_hbm, out_hbm, idx_v, ones_v, acc):
    cid = lax.axis_index("c")
    sid = lax.axis_index("s")
    wid = sid * NC + cid
    pltpu.sync_copy(dst_hbm.at[wid], idx_v)
    pltpu.sync_copy(ones_hbm, ones_v)
    pltpu.sync_copy(zeros_hbm.at[pl.ds(sid * SB, SB)],
                    acc.at[pl.ds(sid * SB, SB)])
    plsc.subcore_barrier()
    for j in range(EC):
        pltpu.sync_copy(ones_v, acc.at[idx_v.at[j]], add=True)
    plsc.subcore_barrier()
    pltpu.sync_copy(acc.at[pl.ds(sid * SB, SB)],
                    out_hbm.at[cid, pl.ds(sid * SB, SB)])


@functools.partial(
    pl.kernel,
    out_type=jax.ShapeDtypeStruct((NC, NPAD, D_IN), jnp.float32),
    mesh=_mesh,
    scratch_types=[
        pltpu.VMEM((4, 2, CHUNK), jnp.int32),
        pltpu.VMEM((2, CHUNK, D_IN), jnp.float32),
        pltpu.VMEM_SHARED((NPAD, D_IN), jnp.float32),
        pltpu.SemaphoreType.DMA,
        pltpu.SemaphoreType.DMA,
        pltpu.SemaphoreType.DMA,
        pltpu.SemaphoreType.DMA,
        pltpu.SemaphoreType.DMA,
        pltpu.SemaphoreType.DMA,
        pltpu.SemaphoreType.DMA,
        pltpu.SemaphoreType.DMA,
    ],
)
def _agg_kernel(y_hbm, idx_hbm, zeros_hbm, out_hbm, idx_v, rows_v, acc,
                si0, si1, si2, si3, sg0, sg1, ss0, ss1):
    cid = lax.axis_index("c")
    sid = lax.axis_index("s")
    wid = sid * NC + cid
    pltpu.sync_copy(zeros_hbm.at[pl.ds(sid * SB, SB)],
                    acc.at[pl.ds(sid * SB, SB)])
    plsc.subcore_barrier()
    semi = (si0, si1, si2, si3)
    semg = (sg0, sg1)
    sems = (ss0, ss1)
    di, dg, ds = {}, {}, {}
    for k in range(min(3, EC)):
        di[k] = pltpu.async_copy(idx_hbm.at[wid, k], idx_v.at[k], semi[k])
    di[0].wait()
    dg[0] = pltpu.async_copy(y_hbm.at[idx_v.at[0, 0]], rows_v.at[0], semg[0])
    for j in range(EC):
        b = j % 2
        nb = (j + 1) % 2
        dg[j].wait()
        ds[j] = pltpu.async_copy(rows_v.at[b], acc.at[idx_v.at[j % 4, 1]],
                                 sems[b], add=True)
        if j + 1 < EC:
            if j >= 1:
                ds[j - 1].wait()
            di[j + 1].wait()
            dg[j + 1] = pltpu.async_copy(
                y_hbm.at[idx_v.at[(j + 1) % 4, 0]], rows_v.at[nb], semg[nb])
            if j + 3 < EC:
                di[j + 3] = pltpu.async_copy(
                    idx_hbm.at[wid, j + 3], idx_v.at[(j + 3) % 4],
                    semi[(j + 3) % 4])
    if EC >= 2:
        ds[EC - 2].wait()
    ds[EC - 1].wait()
    plsc.subcore_barrier()
    pltpu.sync_copy(acc.at[pl.ds(sid * SB, SB)],
                    out_hbm.at[cid, pl.ds(sid * SB, SB)])


def _scale_body(x_ref, degp_ref, y_ref, dinv_ref):
    bm = y_ref.shape[0]
    deg = degp_ref[0, :, 0:1] + degp_ref[1, :, 0:1] + 1.0
    dinv = lax.rsqrt(deg)
    dinv_ref[...] = dinv
    rows = pl.program_id(0) * bm + lax.broadcasted_iota(jnp.int32, (bm, 1), 0)
    y_ref[...] = jnp.where(rows < N_NODES, x_ref[...] * dinv, 0.0)


def _scale_call(x, degp):
    bm = 1024
    return pl.pallas_call(
        _scale_body,
        grid=(NPAD // bm,),
        in_specs=[
            pl.BlockSpec((bm, D_IN), lambda m: (m, 0)),
            pl.BlockSpec((NC, bm, 128), lambda m: (0, m, 0)),
        ],
        out_specs=[
            pl.BlockSpec((bm, D_IN), lambda m: (m, 0)),
            pl.BlockSpec((bm, 1), lambda m: (m, 0)),
        ],
        out_shape=[
            jax.ShapeDtypeStruct((NPAD, D_IN), jnp.float32),
            jax.ShapeDtypeStruct((NPAD, 1), jnp.float32),
        ],
    )(x, degp)


def _head_body(zp_ref, y_ref, dinv_ref, w1_ref, b1_ref, w2_ref, b2_ref, out_ref):
    agg = dinv_ref[...] * (zp_ref[0] + zp_ref[1] + y_ref[...])
    h = jnp.dot(agg, w1_ref[...], preferred_element_type=jnp.float32)
    h = jnp.maximum(h + b1_ref[...], 0.0)
    out = jnp.dot(h, w2_ref[...], preferred_element_type=jnp.float32)
    out_ref[...] = out + b2_ref[...]


def _head_call(zp, y, dinv, w1p, b1p, w2p, b2p):
    bm = 1024
    grid = (NPAD // bm,)
    return pl.pallas_call(
        _head_body,
        grid=grid,
        in_specs=[
            pl.BlockSpec((NC, bm, D_IN), lambda m: (0, m, 0)),
            pl.BlockSpec((bm, D_IN), lambda m: (m, 0)),
            pl.BlockSpec((bm, 1), lambda m: (m, 0)),
            pl.BlockSpec((D_IN, D_HID_PAD), lambda m: (0, 0)),
            pl.BlockSpec((1, D_HID_PAD), lambda m: (0, 0)),
            pl.BlockSpec((D_HID_PAD, D_OUT_PAD), lambda m: (0, 0)),
            pl.BlockSpec((1, D_OUT_PAD), lambda m: (0, 0)),
        ],
        out_specs=pl.BlockSpec((bm, D_OUT_PAD), lambda m: (m, 0)),
        out_shape=jax.ShapeDtypeStruct((NPAD, D_OUT_PAD), jnp.float32),
    )(zp, y, dinv, w1p, b1p, w2p, b2p)


def kernel(x, edge_index, W1, b1, W2, b2):
    x = x.astype(jnp.float32)
    src = edge_index[0].astype(jnp.int32)
    dst = edge_index[1].astype(jnp.int32)
    e = src.shape[0]
    n_pad_e = TOT_E - e
    pad_idx = N_NODES + (jnp.arange(n_pad_e, dtype=jnp.int32) % (NPAD - N_NODES))
    srcp = jnp.concatenate([src, pad_idx]).reshape(NW, EC, 1, CHUNK)
    dstp = jnp.concatenate([dst, pad_idx]).reshape(NW, EC, 1, CHUNK)
    idxp = jnp.concatenate([srcp, dstp], axis=2)

    ones_upd = jnp.ones((CHUNK, 128), jnp.float32)
    zeros_d = jnp.zeros((NPAD, D_IN), jnp.float32)

    degp = _deg_kernel(dstp.reshape(NW, EC, CHUNK), ones_upd, zeros_d)
    y, dinv = _scale_call(x, degp)
    zp = _agg_kernel(y, idxp, zeros_d)

    w1p = jnp.pad(W1.astype(jnp.float32), ((0, 0), (0, D_HID_PAD - D_HID)))
    b1p = jnp.pad(b1.astype(jnp.float32), (0, D_HID_PAD - D_HID)).reshape(1, -1)
    w2p = jnp.pad(W2.astype(jnp.float32),
                  ((0, D_HID_PAD - D_HID), (0, D_OUT_PAD - D_OUT)))
    b2p = jnp.pad(b2.astype(jnp.float32), (0, D_OUT_PAD - D_OUT)).reshape(1, -1)

    out = _head_call(zp, y, dinv, w1p, b1p, w2p, b2p)
    return out[:N_NODES, :D_OUT]


# speedup vs baseline: 1.0028x; 1.0028x over previous
import functools

import jax
import jax.numpy as jnp
from jax import lax
from jax.experimental import pallas as pl
from jax.experimental.pallas import tpu as pltpu
from jax.experimental.pallas import tpu_sc as plsc

N_NODES = 10000
D_IN = 128
D_HID = 300
D_OUT = 2

NPAD = 10240
NC = 2
NS = 16
NW = NC * NS
CHUNK = 128
EC = 79
TOT_E = NW * EC * CHUNK
SB = NPAD // NS

D_HID_PAD = 384
D_OUT_PAD = 128

_mesh = plsc.VectorSubcoreMesh(core_axis_name="c", subcore_axis_name="s")


@functools.partial(
    pl.kernel,
    out_type=jax.ShapeDtypeStruct((NC, NPAD, 128), jnp.float32),
    mesh=_mesh,
    scratch_types=[
        pltpu.VMEM((EC, CHUNK), jnp.int32),
        pltpu.VMEM((CHUNK, 128), jnp.float32),
        pltpu.VMEM_SHARED((NPAD, 128), jnp.float32),
        pltpu.SemaphoreType.DMA,
    ],
)
def _deg_kernel(dst_hbm, ones_hbm, zeros_hbm, out_hbm, idx_v, ones_v, acc, sem):
    cid = lax.axis_index("c")
    sid = lax.axis_index("s")
    wid = sid * NC + cid
    pltpu.sync_copy(dst_hbm.at[wid], idx_v)
    pltpu.sync_copy(ones_hbm, ones_v)
    pltpu.sync_copy(zeros_hbm.at[pl.ds(sid * SB, SB)],
                    acc.at[pl.ds(sid * SB, SB)])
    plsc.subcore_barrier()
    ds = {}
    for j in range(EC):
        ds[j] = pltpu.async_copy(ones_v, acc.at[idx_v.at[j]], sem, add=True)
    for j in range(EC):
        ds[j].wait()
    plsc.subcore_barrier()
    pltpu.sync_copy(acc.at[pl.ds(sid * SB, SB)],
                    out_hbm.at[cid, pl.ds(sid * SB, SB)])


@functools.partial(
    pl.kernel,
    out_type=jax.ShapeDtypeStruct((NC, NPAD, D_IN), jnp.float32),
    mesh=_mesh,
    scratch_types=[
        pltpu.VMEM((4, 2, CHUNK), jnp.int32),
        pltpu.VMEM((2, CHUNK, D_IN), jnp.float32),
        pltpu.VMEM_SHARED((NPAD, D_IN), jnp.float32),
        pltpu.SemaphoreType.DMA,
        pltpu.SemaphoreType.DMA,
        pltpu.SemaphoreType.DMA,
        pltpu.SemaphoreType.DMA,
        pltpu.SemaphoreType.DMA,
        pltpu.SemaphoreType.DMA,
        pltpu.SemaphoreType.DMA,
        pltpu.SemaphoreType.DMA,
    ],
)
def _agg_kernel(y_hbm, idx_hbm, zeros_hbm, out_hbm, idx_v, rows_v, acc,
                si0, si1, si2, si3, sg0, sg1, ss0, ss1):
    cid = lax.axis_index("c")
    sid = lax.axis_index("s")
    wid = sid * NC + cid
    pltpu.sync_copy(zeros_hbm.at[pl.ds(sid * SB, SB)],
                    acc.at[pl.ds(sid * SB, SB)])
    plsc.subcore_barrier()
    semi = (si0, si1, si2, si3)
    semg = (sg0, sg1)
    sems = (ss0, ss1)
    di, dg, ds = {}, {}, {}
    for k in range(min(3, EC)):
        di[k] = pltpu.async_copy(idx_hbm.at[wid, k], idx_v.at[k], semi[k])
    di[0].wait()
    dg[0] = pltpu.async_copy(y_hbm.at[idx_v.at[0, 0]], rows_v.at[0], semg[0])
    for j in range(EC):
        b = j % 2
        nb = (j + 1) % 2
        dg[j].wait()
        ds[j] = pltpu.async_copy(rows_v.at[b], acc.at[idx_v.at[j % 4, 1]],
                                 sems[b], add=True)
        if j + 1 < EC:
            if j >= 1:
                ds[j - 1].wait()
            di[j + 1].wait()
            dg[j + 1] = pltpu.async_copy(
                y_hbm.at[idx_v.at[(j + 1) % 4, 0]], rows_v.at[nb], semg[nb])
            if j + 3 < EC:
                di[j + 3] = pltpu.async_copy(
                    idx_hbm.at[wid, j + 3], idx_v.at[(j + 3) % 4],
                    semi[(j + 3) % 4])
    if EC >= 2:
        ds[EC - 2].wait()
    ds[EC - 1].wait()
    plsc.subcore_barrier()
    pltpu.sync_copy(acc.at[pl.ds(sid * SB, SB)],
                    out_hbm.at[cid, pl.ds(sid * SB, SB)])


def _scale_body(x_ref, degp_ref, y_ref, dinv_ref):
    bm = y_ref.shape[0]
    deg = degp_ref[0, :, 0:1] + degp_ref[1, :, 0:1] + 1.0
    dinv = lax.rsqrt(deg)
    dinv_ref[...] = dinv
    rows = pl.program_id(0) * bm + lax.broadcasted_iota(jnp.int32, (bm, 1), 0)
    y_ref[...] = jnp.where(rows < N_NODES, x_ref[...] * dinv, 0.0)


def _scale_call(x, degp):
    bm = 1024
    return pl.pallas_call(
        _scale_body,
        grid=(NPAD // bm,),
        in_specs=[
            pl.BlockSpec((bm, D_IN), lambda m: (m, 0)),
            pl.BlockSpec((NC, bm, 128), lambda m: (0, m, 0)),
        ],
        out_specs=[
            pl.BlockSpec((bm, D_IN), lambda m: (m, 0)),
            pl.BlockSpec((bm, 1), lambda m: (m, 0)),
        ],
        out_shape=[
            jax.ShapeDtypeStruct((NPAD, D_IN), jnp.float32),
            jax.ShapeDtypeStruct((NPAD, 1), jnp.float32),
        ],
    )(x, degp)


def _head_body(zp_ref, y_ref, dinv_ref, w1_ref, b1_ref, w2_ref, b2_ref, out_ref):
    agg = dinv_ref[...] * (zp_ref[0] + zp_ref[1] + y_ref[...])
    h = jnp.dot(agg, w1_ref[...], preferred_element_type=jnp.float32)
    h = jnp.maximum(h + b1_ref[...], 0.0)
    out = jnp.dot(h, w2_ref[...], preferred_element_type=jnp.float32)
    out_ref[...] = out + b2_ref[...]


def _head_call(zp, y, dinv, w1p, b1p, w2p, b2p):
    bm = 1024
    grid = (NPAD // bm,)
    return pl.pallas_call(
        _head_body,
        grid=grid,
        in_specs=[
            pl.BlockSpec((NC, bm, D_IN), lambda m: (0, m, 0)),
            pl.BlockSpec((bm, D_IN), lambda m: (m, 0)),
            pl.BlockSpec((bm, 1), lambda m: (m, 0)),
            pl.BlockSpec((D_IN, D_HID_PAD), lambda m: (0, 0)),
            pl.BlockSpec((1, D_HID_PAD), lambda m: (0, 0)),
            pl.BlockSpec((D_HID_PAD, D_OUT_PAD), lambda m: (0, 0)),
            pl.BlockSpec((1, D_OUT_PAD), lambda m: (0, 0)),
        ],
        out_specs=pl.BlockSpec((bm, D_OUT_PAD), lambda m: (m, 0)),
        out_shape=jax.ShapeDtypeStruct((NPAD, D_OUT_PAD), jnp.float32),
    )(zp, y, dinv, w1p, b1p, w2p, b2p)


def kernel(x, edge_index, W1, b1, W2, b2):
    x = x.astype(jnp.float32)
    src = edge_index[0].astype(jnp.int32)
    dst = edge_index[1].astype(jnp.int32)
    e = src.shape[0]
    n_pad_e = TOT_E - e
    pad_idx = N_NODES + (jnp.arange(n_pad_e, dtype=jnp.int32) % (NPAD - N_NODES))
    srcp = jnp.concatenate([src, pad_idx]).reshape(NW, EC, 1, CHUNK)
    dstp = jnp.concatenate([dst, pad_idx]).reshape(NW, EC, 1, CHUNK)
    idxp = jnp.concatenate([srcp, dstp], axis=2)

    ones_upd = jnp.ones((CHUNK, 128), jnp.float32)
    zeros_d = jnp.zeros((NPAD, D_IN), jnp.float32)

    degp = _deg_kernel(dstp.reshape(NW, EC, CHUNK), ones_upd, zeros_d)
    y, dinv = _scale_call(x, degp)
    zp = _agg_kernel(y, idxp, zeros_d)

    w1p = jnp.pad(W1.astype(jnp.float32), ((0, 0), (0, D_HID_PAD - D_HID)))
    b1p = jnp.pad(b1.astype(jnp.float32), (0, D_HID_PAD - D_HID)).reshape(1, -1)
    w2p = jnp.pad(W2.astype(jnp.float32),
                  ((0, D_HID_PAD - D_HID), (0, D_OUT_PAD - D_OUT)))
    b2p = jnp.pad(b2.astype(jnp.float32), (0, D_OUT_PAD - D_OUT)).reshape(1, -1)

    out = _head_call(zp, y, dinv, w1p, b1p, w2p, b2p)
    return out[:N_NODES, :D_OUT]
